# baseline (speedup 1.0000x reference)
import jax
import jax.numpy as jnp
from jax.experimental import pallas as pl
from jax.experimental.pallas import tpu as pltpu

_BB = 16


def _onehot_block(idx_ref, out_ref):
    idx = idx_ref[:, :]
    col = jax.lax.broadcasted_iota(jnp.int32, out_ref.shape, 1)
    out_ref[:, :] = (col == idx).astype(jnp.float32)


def kernel(indexes, weight):
    vocab = weight.shape[0]
    batch = indexes.shape[0]
    idx2 = indexes.astype(jnp.int32).reshape(batch, 1)
    return pl.pallas_call(
        _onehot_block,
        out_shape=jax.ShapeDtypeStruct((batch, vocab), jnp.float32),
        grid=(batch // _BB,),
        in_specs=[pl.BlockSpec((_BB, 1), lambda i: (i, 0))],
        out_specs=pl.BlockSpec((_BB, vocab), lambda i: (i, 0)),
        compiler_params=pltpu.CompilerParams(
            dimension_semantics=("parallel",),
        ),
    )(idx2)


# speedup vs baseline: 1.0010x; 1.0010x over previous
import jax
import jax.numpy as jnp
from jax.experimental import pallas as pl
from jax.experimental.pallas import tpu as pltpu

_BB = 16


def _onehot_dma(idx_ref, out_ref, buf_ref, sem0, sem1):
    j = pl.program_id(0)
    nsteps = pl.num_programs(0)
    slot = jax.lax.rem(j, 2)

    @pl.when(j == 0)
    def _init():
        buf_ref[...] = jnp.zeros_like(buf_ref)

    sem = [sem0, sem1]

    @pl.when(j >= 2)
    def _recycle():
        for s in range(2):
            @pl.when(slot == s)
            def _():
                pltpu.make_async_copy(
                    buf_ref.at[s],
                    out_ref.at[pl.ds((j - 2) * _BB, _BB), :],
                    sem[s]).wait()
        for i in range(_BB):
            c = idx_ref[(j - 2) * _BB + i]
            base = pl.multiple_of((c // 128) * 128, 128)
            buf_ref[slot, i, pl.ds(base, 128)] = jnp.zeros((128,), jnp.float32)

    for i in range(_BB):
        c = idx_ref[j * _BB + i]
        base = pl.multiple_of((c // 128) * 128, 128)
        lane = jax.lax.broadcasted_iota(jnp.int32, (128,), 0)
        vec = (lane == (c - base)).astype(jnp.float32)
        buf_ref[slot, i, pl.ds(base, 128)] = vec

    for s in range(2):
        @pl.when(slot == s)
        def _():
            pltpu.make_async_copy(
                buf_ref.at[s],
                out_ref.at[pl.ds(j * _BB, _BB), :],
                sem[s]).start()

    @pl.when(j == nsteps - 1)
    def _drain():
        last = (nsteps - 1) % 2
        other = 1 - last
        pltpu.make_async_copy(
            buf_ref.at[other],
            out_ref.at[pl.ds((nsteps - 2) * _BB, _BB), :],
            sem[other]).wait()
        pltpu.make_async_copy(
            buf_ref.at[last],
            out_ref.at[pl.ds((nsteps - 1) * _BB, _BB), :],
            sem[last]).wait()


def kernel(indexes, weight):
    vocab = weight.shape[0]
    batch = indexes.shape[0]
    idx = indexes.astype(jnp.int32)
    grid_spec = pltpu.PrefetchScalarGridSpec(
        num_scalar_prefetch=1,
        grid=(batch // _BB,),
        in_specs=[],
        out_specs=pl.BlockSpec(memory_space=pl.ANY),
        scratch_shapes=[
            pltpu.VMEM((2, _BB, vocab), jnp.float32),
            pltpu.SemaphoreType.DMA,
            pltpu.SemaphoreType.DMA,
        ],
    )
    return pl.pallas_call(
        _onehot_dma,
        grid_spec=grid_spec,
        out_shape=jax.ShapeDtypeStruct((batch, vocab), jnp.float32),
    )(idx)


# speedup vs baseline: 1.0041x; 1.0031x over previous
import jax
import jax.numpy as jnp
from jax.experimental import pallas as pl
from jax.experimental.pallas import tpu as pltpu

_BB = 16
_NSLOT = 4


def _onehot_dma(idx_ref, out_ref, buf_ref, *sems):
    j = pl.program_id(0)
    nsteps = pl.num_programs(0)
    slot = jax.lax.rem(j, _NSLOT)

    @pl.when(j == 0)
    def _init():
        buf_ref[...] = jnp.zeros_like(buf_ref)

    def poke(row_ref, c, value):
        base = pl.multiple_of((c // 128) * 128, 128)
        lane = jax.lax.broadcasted_iota(jnp.int32, (128,), 0)
        vec = jnp.where(lane == (c - base), value, 0.0).astype(jnp.float32)
        row_ref[pl.ds(base, 128)] = vec

    @pl.when(j >= _NSLOT)
    def _recycle():
        for s in range(_NSLOT):
            @pl.when(slot == s)
            def _():
                pltpu.make_async_copy(
                    buf_ref.at[s],
                    out_ref.at[pl.ds((j - _NSLOT) * _BB, _BB), :],
                    sems[s]).wait()
        for i in range(_BB):
            c = idx_ref[(j - _NSLOT) * _BB + i]
            poke(buf_ref.at[slot, i], c, 0.0)

    for i in range(_BB):
        c = idx_ref[j * _BB + i]
        poke(buf_ref.at[slot, i], c, 1.0)

    for s in range(_NSLOT):
        @pl.when(slot == s)
        def _():
            pltpu.make_async_copy(
                buf_ref.at[s],
                out_ref.at[pl.ds(j * _BB, _BB), :],
                sems[s]).start()

    @pl.when(j == nsteps - 1)
    def _drain():
        for t in range(_NSLOT):
            step = nsteps - _NSLOT + t
            pltpu.make_async_copy(
                buf_ref.at[step % _NSLOT],
                out_ref.at[pl.ds(step * _BB, _BB), :],
                sems[step % _NSLOT]).wait()


def kernel(indexes, weight):
    vocab = weight.shape[0]
    batch = indexes.shape[0]
    idx = indexes.astype(jnp.int32)
    grid_spec = pltpu.PrefetchScalarGridSpec(
        num_scalar_prefetch=1,
        grid=(batch // _BB,),
        in_specs=[],
        out_specs=pl.BlockSpec(memory_space=pl.ANY),
        scratch_shapes=[
            pltpu.VMEM((_NSLOT, _BB, vocab), jnp.float32),
        ] + [pltpu.SemaphoreType.DMA] * _NSLOT,
    )
    return pl.pallas_call(
        _onehot_dma,
        grid_spec=grid_spec,
        out_shape=jax.ShapeDtypeStruct((batch, vocab), jnp.float32),
    )(idx)


# speedup vs baseline: 1.0366x; 1.0324x over previous
import jax
import jax.numpy as jnp
from jax.experimental import pallas as pl
from jax.experimental.pallas import tpu as pltpu

_BB = 4
_NSLOT = 8


def _onehot_dma(idx_ref, out_ref, buf_ref, *sems):
    j = pl.program_id(0)
    nsteps = pl.num_programs(0)
    slot = jax.lax.rem(j, _NSLOT)

    @pl.when(j == 0)
    def _init():
        buf_ref[...] = jnp.zeros_like(buf_ref)

    def poke(row_ref, c, value):
        base = pl.multiple_of((c // 128) * 128, 128)
        lane = jax.lax.broadcasted_iota(jnp.int32, (128,), 0)
        vec = jnp.where(lane == (c - base), value, 0.0).astype(jnp.float32)
        row_ref[pl.ds(base, 128)] = vec

    @pl.when(j >= _NSLOT)
    def _recycle():
        for s in range(_NSLOT):
            @pl.when(slot == s)
            def _():
                pltpu.make_async_copy(
                    buf_ref.at[s],
                    out_ref.at[pl.ds((j - _NSLOT) * _BB, _BB), :],
                    sems[s]).wait()
        for i in range(_BB):
            c = idx_ref[(j - _NSLOT) * _BB + i]
            poke(buf_ref.at[slot, i], c, 0.0)

    for i in range(_BB):
        c = idx_ref[j * _BB + i]
        poke(buf_ref.at[slot, i], c, 1.0)

    for s in range(_NSLOT):
        @pl.when(slot == s)
        def _():
            pltpu.make_async_copy(
                buf_ref.at[s],
                out_ref.at[pl.ds(j * _BB, _BB), :],
                sems[s]).start()

    @pl.when(j == nsteps - 1)
    def _drain():
        for t in range(_NSLOT):
            step = nsteps - _NSLOT + t
            pltpu.make_async_copy(
                buf_ref.at[step % _NSLOT],
                out_ref.at[pl.ds(step * _BB, _BB), :],
                sems[step % _NSLOT]).wait()


def kernel(indexes, weight):
    vocab = weight.shape[0]
    batch = indexes.shape[0]
    idx = indexes.astype(jnp.int32)
    grid_spec = pltpu.PrefetchScalarGridSpec(
        num_scalar_prefetch=1,
        grid=(batch // _BB,),
        in_specs=[],
        out_specs=pl.BlockSpec(memory_space=pl.ANY),
        scratch_shapes=[
            pltpu.VMEM((_NSLOT, _BB, vocab), jnp.float32),
        ] + [pltpu.SemaphoreType.DMA] * _NSLOT,
    )
    return pl.pallas_call(
        _onehot_dma,
        grid_spec=grid_spec,
        out_shape=jax.ShapeDtypeStruct((batch, vocab), jnp.float32),
    )(idx)
